# Initial kernel scaffold; baseline (speedup 1.0000x reference)
import jax
import jax.numpy as jnp
from jax.experimental import pallas as pl


def kernel(features_2d, projection_indices):
    raise NotImplementedError("write your pallas kernel here")


# speedup vs baseline: 8.2916x; 8.2916x over previous
import jax
import jax.numpy as jnp
from jax import lax
from jax.experimental import pallas as pl
from jax.experimental.pallas import tpu as pltpu
from jax.experimental.pallas import tpu_sc as plsc

B, C, H, W = 2, 96, 384, 384
VOX = 128 * 128 * 16
NPIX = H * W
NCH = B * C
NC, NS = 2, 16
CH_PER_SC = NCH // NC
PIX_PER_TILE = NPIX // NS
CHUNK = 128
NCHUNK = PIX_PER_TILE // CHUNK
VOX_PER_TILE = VOX // NS


def _sc_scatter(feat_hbm, idx_hbm, out_hbm, vals, idx2d, zeros, grid):
    c = lax.axis_index("c")
    s = lax.axis_index("s")

    pltpu.sync_copy(idx_hbm.at[pl.ds(s * NCHUNK, NCHUNK), :], idx2d)

    def _zero_body(k, _):
        zeros[pl.ds(k * 16, 16)] = jnp.zeros((16,), jnp.float32)
        return 0
    lax.fori_loop(0, VOX_PER_TILE // 16, _zero_body, 0)
    pltpu.sync_copy(zeros, grid.at[pl.ds(s * VOX_PER_TILE, VOX_PER_TILE)])
    plsc.subcore_barrier()

    def _channel_body(t, _):
        ch = c * CH_PER_SC + t
        pltpu.sync_copy(feat_hbm.at[ch, pl.ds(s * NCHUNK, NCHUNK), :], vals)

        def _chunk_body(j, _):
            pltpu.sync_copy(vals.at[j], grid.at[idx2d.at[j]], add=True)
            return 0
        lax.fori_loop(0, NCHUNK, _chunk_body, 0)
        plsc.subcore_barrier()

        v0 = s * VOX_PER_TILE
        pltpu.sync_copy(grid.at[pl.ds(v0, VOX_PER_TILE)],
                        out_hbm.at[ch, pl.ds(v0, VOX_PER_TILE)])
        pltpu.sync_copy(zeros, grid.at[pl.ds(v0, VOX_PER_TILE)])
        plsc.subcore_barrier()
        return 0

    lax.fori_loop(0, CH_PER_SC, _channel_body, 0)


@jax.jit
def kernel(features_2d, projection_indices):
    feat = features_2d.reshape(NCH, NPIX // 128, 128)
    idx = projection_indices.reshape(NPIX // 128, 128)

    mesh = plsc.VectorSubcoreMesh(core_axis_name="c", subcore_axis_name="s")
    run = pl.kernel(
        _sc_scatter,
        mesh=mesh,
        out_type=jax.ShapeDtypeStruct((NCH, VOX), jnp.float32),
        scratch_types=[
            pltpu.VMEM((NCHUNK, CHUNK), jnp.float32),
            pltpu.VMEM((NCHUNK, CHUNK), jnp.int32),
            pltpu.VMEM((VOX_PER_TILE,), jnp.float32),
            pltpu.VMEM_SHARED((VOX,), jnp.float32),
        ],
    )
    out = run(feat, idx)
    return out.reshape(B, C, 128, 128, 16)


# speedup vs baseline: 11.9319x; 1.4390x over previous
import jax
import jax.numpy as jnp
from jax import lax
from jax.experimental import pallas as pl
from jax.experimental.pallas import tpu as pltpu
from jax.experimental.pallas import tpu_sc as plsc

B, C, H, W = 2, 96, 384, 384
VOX = 128 * 128 * 16
NPIX = H * W
NCH = B * C
NC, NS = 2, 16
CH_PER_SC = NCH // NC
PIX_PER_TILE = NPIX // NS
CHUNK = 128
NCHUNK = PIX_PER_TILE // CHUNK
VOX_PER_TILE = VOX // NS


def _sc_scatter(feat_hbm, idx_hbm, out_hbm, vals_a, vals_b, idx1d, zeros,
                grid, sem_a, sem_b):
    c = lax.axis_index("c")
    s = lax.axis_index("s")
    rows = pl.ds(s * PIX_PER_TILE, PIX_PER_TILE)
    ch_lo = c * CH_PER_SC
    ch_hi = ch_lo + CH_PER_SC - 1

    pltpu.sync_copy(idx_hbm.at[rows], idx1d)

    def _zero_body(k, _):
        zeros[pl.ds(k * 16, 16)] = jnp.zeros((16,), jnp.float32)
        return 0
    lax.fori_loop(0, VOX_PER_TILE // 16, _zero_body, 0)
    v0 = s * VOX_PER_TILE
    gslice = grid.at[pl.ds(v0, VOX_PER_TILE)]
    pltpu.sync_copy(zeros, gslice)

    pltpu.async_copy(feat_hbm.at[ch_lo, rows], vals_a, sem_a)
    plsc.subcore_barrier()

    def _do_channel(ch, vals, sem, nxt_ch, nxt_vals, nxt_sem):
        pltpu.make_async_copy(feat_hbm.at[ch, rows], vals, sem).wait()
        pltpu.async_copy(feat_hbm.at[nxt_ch, rows], nxt_vals, nxt_sem)
        pltpu.sync_copy(vals, grid.at[idx1d], add=True)
        plsc.subcore_barrier()
        pltpu.sync_copy(gslice, out_hbm.at[ch, pl.ds(v0, VOX_PER_TILE)])
        pltpu.sync_copy(zeros, gslice)
        plsc.subcore_barrier()

    def _pair_body(i, _):
        ch0 = ch_lo + 2 * i
        ch1 = ch0 + 1
        _do_channel(ch0, vals_a, sem_a, ch1, vals_b, sem_b)
        _do_channel(ch1, vals_b, sem_b, jnp.minimum(ch0 + 2, ch_hi),
                    vals_a, sem_a)
        return 0

    lax.fori_loop(0, CH_PER_SC // 2, _pair_body, 0)
    pltpu.make_async_copy(feat_hbm.at[ch_hi, rows], vals_a, sem_a).wait()


@jax.jit
def kernel(features_2d, projection_indices):
    feat = features_2d.reshape(NCH, NPIX)
    idx = projection_indices.reshape(NPIX)

    mesh = plsc.VectorSubcoreMesh(core_axis_name="c", subcore_axis_name="s")
    run = pl.kernel(
        _sc_scatter,
        mesh=mesh,
        out_type=jax.ShapeDtypeStruct((NCH, VOX), jnp.float32),
        scratch_types=[
            pltpu.VMEM((PIX_PER_TILE,), jnp.float32),
            pltpu.VMEM((PIX_PER_TILE,), jnp.float32),
            pltpu.VMEM((PIX_PER_TILE,), jnp.int32),
            pltpu.VMEM((VOX_PER_TILE,), jnp.float32),
            pltpu.VMEM_SHARED((VOX,), jnp.float32),
            pltpu.SemaphoreType.DMA,
            pltpu.SemaphoreType.DMA,
        ],
    )
    out = run(feat, idx)
    return out.reshape(B, C, 128, 128, 16)


# speedup vs baseline: 13.7388x; 1.1514x over previous
import jax
import jax.numpy as jnp
from jax import lax
from jax.experimental import pallas as pl
from jax.experimental.pallas import tpu as pltpu
from jax.experimental.pallas import tpu_sc as plsc

B, C, H, W = 2, 96, 384, 384
VOX = 128 * 128 * 16
NPIX = H * W
NCH = B * C
NC, NS = 2, 16
CH_PER_SC = NCH // NC
PIX_PER_TILE = NPIX // NS
VOX_PER_TILE = VOX // NS


def _sc_scatter(feat_hbm, idx_hbm, out_hbm,
                v0, v1, v2, idx1d, zeros, g0, g1, g2,
                sv0, sv1, sv2, swb0, swb1, swb2, sz0, sz1, sz2):
    c = lax.axis_index("c")
    s = lax.axis_index("s")
    rows = pl.ds(s * PIX_PER_TILE, PIX_PER_TILE)
    ch_lo = c * CH_PER_SC
    ch_hi = ch_lo + CH_PER_SC - 1
    x0 = s * VOX_PER_TILE
    vslice = pl.ds(x0, VOX_PER_TILE)

    vbufs = (v0, v1, v2)
    vsems = (sv0, sv1, sv2)
    grids = (g0, g1, g2)
    wsems = (swb0, swb1, swb2)
    zsems = (sz0, sz1, sz2)

    pltpu.sync_copy(idx_hbm.at[rows], idx1d)

    def _zero_body(k, _):
        zeros[pl.ds(k * 16, 16)] = jnp.zeros((16,), jnp.float32)
        return 0
    lax.fori_loop(0, VOX_PER_TILE // 16, _zero_body, 0)
    for g, sz in zip(grids, zsems):
        pltpu.async_copy(zeros, g.at[vslice], sz)
    pltpu.async_copy(feat_hbm.at[ch_lo, rows], v0, sv0)
    pltpu.async_copy(feat_hbm.at[ch_lo + 1, rows], v1, sv1)

    def _step(ch, k, first):
        vk, svk, gk, szk = vbufs[k], vsems[k], grids[k], zsems[k]
        gh, swbh, szh = grids[(k + 1) % 3], wsems[(k + 1) % 3], zsems[(k + 1) % 3]
        vp, svp = vbufs[(k + 2) % 3], vsems[(k + 2) % 3]
        pltpu.make_async_copy(feat_hbm.at[ch, rows], vk, svk).wait()
        pltpu.async_copy(
            feat_hbm.at[jnp.minimum(ch + 2, ch_hi), rows], vp, svp)
        if not first:
            pltpu.make_async_copy(gh.at[vslice],
                                  out_hbm.at[ch, vslice], swbh).wait()
            pltpu.async_copy(zeros, gh.at[vslice], szh)
        pltpu.make_async_copy(zeros, gk.at[vslice], szk).wait()
        plsc.subcore_barrier()
        pltpu.sync_copy(vk, gk.at[idx1d], add=True)
        plsc.subcore_barrier()
        pltpu.async_copy(gk.at[vslice], out_hbm.at[ch, vslice], wsems[k])

    _step(ch_lo + 0, 0, True)
    _step(ch_lo + 1, 1, True)
    _step(ch_lo + 2, 2, False)

    def _trip_body(i, _):
        ch = ch_lo + 3 + 3 * i
        _step(ch + 0, 0, False)
        _step(ch + 1, 1, False)
        _step(ch + 2, 2, False)
        return 0

    lax.fori_loop(0, (CH_PER_SC - 3) // 3, _trip_body, 0)

    pltpu.make_async_copy(g1.at[vslice], out_hbm.at[ch_hi, vslice], swb1).wait()
    pltpu.make_async_copy(g2.at[vslice], out_hbm.at[ch_hi, vslice], swb2).wait()
    pltpu.make_async_copy(zeros, g0.at[vslice], sz0).wait()
    pltpu.make_async_copy(feat_hbm.at[ch_hi, rows], v0, sv0).wait()
    pltpu.make_async_copy(feat_hbm.at[ch_hi, rows], v1, sv1).wait()


@jax.jit
def kernel(features_2d, projection_indices):
    feat = features_2d.reshape(NCH, NPIX)
    idx = projection_indices.reshape(NPIX)

    mesh = plsc.VectorSubcoreMesh(core_axis_name="c", subcore_axis_name="s")
    run = pl.kernel(
        _sc_scatter,
        mesh=mesh,
        out_type=jax.ShapeDtypeStruct((NCH, VOX), jnp.float32),
        scratch_types=[
            pltpu.VMEM((PIX_PER_TILE,), jnp.float32),
            pltpu.VMEM((PIX_PER_TILE,), jnp.float32),
            pltpu.VMEM((PIX_PER_TILE,), jnp.float32),
            pltpu.VMEM((PIX_PER_TILE,), jnp.int32),
            pltpu.VMEM((VOX_PER_TILE,), jnp.float32),
            pltpu.VMEM_SHARED((VOX,), jnp.float32),
            pltpu.VMEM_SHARED((VOX,), jnp.float32),
            pltpu.VMEM_SHARED((VOX,), jnp.float32),
            pltpu.SemaphoreType.DMA,
            pltpu.SemaphoreType.DMA,
            pltpu.SemaphoreType.DMA,
            pltpu.SemaphoreType.DMA,
            pltpu.SemaphoreType.DMA,
            pltpu.SemaphoreType.DMA,
            pltpu.SemaphoreType.DMA,
            pltpu.SemaphoreType.DMA,
            pltpu.SemaphoreType.DMA,
        ],
    )
    out = run(feat, idx)
    return out.reshape(B, C, 128, 128, 16)


# speedup vs baseline: 13.7541x; 1.0011x over previous
import jax
import jax.numpy as jnp
from jax import lax
from jax.experimental import pallas as pl
from jax.experimental.pallas import tpu as pltpu
from jax.experimental.pallas import tpu_sc as plsc

B, C, H, W = 2, 96, 384, 384
VOX = 128 * 128 * 16
NPIX = H * W
NCH = B * C
NC, NS = 2, 16
CH_PER_SC = NCH // NC
PIX_PER_TILE = NPIX // NS
VOX_PER_TILE = VOX // NS


def _sc_scatter(feat_hbm, idx_hbm, out_hbm,
                v0, v1, v2, idxA, idxB, zeros, g0, g1, g2,
                sv0, sv1, sv2, swb0, swb1, swb2, sz0, sz1, sz2, ssc):
    c = lax.axis_index("c")
    s = lax.axis_index("s")
    rows = pl.ds(s * PIX_PER_TILE, PIX_PER_TILE)
    ch_lo = c * CH_PER_SC
    ch_hi = ch_lo + CH_PER_SC - 1
    x0 = s * VOX_PER_TILE
    vslice = pl.ds(x0, VOX_PER_TILE)

    vbufs = (v0, v1, v2)
    vsems = (sv0, sv1, sv2)
    grids = (g0, g1, g2)
    wsems = (swb0, swb1, swb2)
    zsems = (sz0, sz1, sz2)

    half = PIX_PER_TILE // 2
    pltpu.sync_copy(idx_hbm.at[pl.ds(s * PIX_PER_TILE, half)], idxA)
    pltpu.sync_copy(idx_hbm.at[pl.ds(s * PIX_PER_TILE + half, half)], idxB)

    def _zero_body(k, _):
        zeros[pl.ds(k * 16, 16)] = jnp.zeros((16,), jnp.float32)
        return 0
    lax.fori_loop(0, VOX_PER_TILE // 16, _zero_body, 0)
    for g, sz in zip(grids, zsems):
        pltpu.async_copy(zeros, g.at[vslice], sz)
    pltpu.async_copy(feat_hbm.at[ch_lo, rows], v0, sv0)
    pltpu.async_copy(feat_hbm.at[ch_lo + 1, rows], v1, sv1)

    def _step(ch, k, first):
        vk, svk, gk, szk = vbufs[k], vsems[k], grids[k], zsems[k]
        gh, swbh, szh = grids[(k + 1) % 3], wsems[(k + 1) % 3], zsems[(k + 1) % 3]
        vp, svp = vbufs[(k + 2) % 3], vsems[(k + 2) % 3]
        pltpu.make_async_copy(feat_hbm.at[ch, rows], vk, svk).wait()
        pltpu.async_copy(
            feat_hbm.at[jnp.minimum(ch + 2, ch_hi), rows], vp, svp)
        if not first:
            pltpu.make_async_copy(gh.at[vslice],
                                  out_hbm.at[ch, vslice], swbh).wait()
            pltpu.async_copy(zeros, gh.at[vslice], szh)
        pltpu.make_async_copy(zeros, gk.at[vslice], szk).wait()
        plsc.subcore_barrier()
        half = PIX_PER_TILE // 2
        pltpu.async_copy(vk.at[pl.ds(0, half)], gk.at[idxA], ssc, add=True)
        pltpu.async_copy(vk.at[pl.ds(half, half)], gk.at[idxB], ssc, add=True)
        pltpu.make_async_copy(vk.at[pl.ds(0, half)], gk.at[idxA], ssc).wait()
        pltpu.make_async_copy(vk.at[pl.ds(half, half)], gk.at[idxB], ssc).wait()
        plsc.subcore_barrier()
        pltpu.async_copy(gk.at[vslice], out_hbm.at[ch, vslice], wsems[k])

    _step(ch_lo + 0, 0, True)
    _step(ch_lo + 1, 1, True)
    _step(ch_lo + 2, 2, False)

    def _trip_body(i, _):
        ch = ch_lo + 3 + 3 * i
        _step(ch + 0, 0, False)
        _step(ch + 1, 1, False)
        _step(ch + 2, 2, False)
        return 0

    lax.fori_loop(0, (CH_PER_SC - 3) // 3, _trip_body, 0)

    pltpu.make_async_copy(g1.at[vslice], out_hbm.at[ch_hi, vslice], swb1).wait()
    pltpu.make_async_copy(g2.at[vslice], out_hbm.at[ch_hi, vslice], swb2).wait()
    pltpu.make_async_copy(zeros, g0.at[vslice], sz0).wait()
    pltpu.make_async_copy(feat_hbm.at[ch_hi, rows], v0, sv0).wait()
    pltpu.make_async_copy(feat_hbm.at[ch_hi, rows], v1, sv1).wait()


@jax.jit
def kernel(features_2d, projection_indices):
    feat = features_2d.reshape(NCH, NPIX)
    idx = projection_indices.reshape(NPIX)

    mesh = plsc.VectorSubcoreMesh(core_axis_name="c", subcore_axis_name="s")
    run = pl.kernel(
        _sc_scatter,
        mesh=mesh,
        out_type=jax.ShapeDtypeStruct((NCH, VOX), jnp.float32),
        scratch_types=[
            pltpu.VMEM((PIX_PER_TILE,), jnp.float32),
            pltpu.VMEM((PIX_PER_TILE,), jnp.float32),
            pltpu.VMEM((PIX_PER_TILE,), jnp.float32),
            pltpu.VMEM((PIX_PER_TILE // 2,), jnp.int32),
            pltpu.VMEM((PIX_PER_TILE // 2,), jnp.int32),
            pltpu.VMEM((VOX_PER_TILE,), jnp.float32),
            pltpu.VMEM_SHARED((VOX,), jnp.float32),
            pltpu.VMEM_SHARED((VOX,), jnp.float32),
            pltpu.VMEM_SHARED((VOX,), jnp.float32),
            pltpu.SemaphoreType.DMA,
            pltpu.SemaphoreType.DMA,
            pltpu.SemaphoreType.DMA,
            pltpu.SemaphoreType.DMA,
            pltpu.SemaphoreType.DMA,
            pltpu.SemaphoreType.DMA,
            pltpu.SemaphoreType.DMA,
            pltpu.SemaphoreType.DMA,
            pltpu.SemaphoreType.DMA,
            pltpu.SemaphoreType.DMA,
        ],
    )
    out = run(feat, idx)
    return out.reshape(B, C, 128, 128, 16)
